# Initial kernel scaffold; baseline (speedup 1.0000x reference)
import jax
import jax.numpy as jnp
from jax.experimental import pallas as pl


def kernel(images_all, indices_class):
    raise NotImplementedError("write your pallas kernel here")


# speedup vs baseline: 2.0894x; 2.0894x over previous
import functools

import jax
import jax.numpy as jnp
from jax import lax
from jax.experimental import pallas as pl
from jax.experimental.pallas import tpu as pltpu
from jax.experimental.pallas import tpu_sc as plsc

_NUM_CLASSES = 10
_BATCH = 256
_N_IMAGES = 50000
_D = 3 * 32 * 32
_B = _NUM_CLASSES * _BATCH

_NC, _NS, _L = 2, 16, 16
_NW = _NC * _NS
_BPW = _B // _NW
_CHUNK = 16
_NCHUNK = _BPW // _CHUNK

_mesh = plsc.VectorSubcoreMesh(core_axis_name="c", subcore_axis_name="s")


@functools.partial(
    pl.kernel,
    out_type=(
        jax.ShapeDtypeStruct((_B, _D), jnp.float32),
        jax.ShapeDtypeStruct((_B,), jnp.int32),
    ),
    mesh=_mesh,
    scratch_types=[
        pltpu.VMEM((_BPW,), jnp.int32),
        pltpu.VMEM((2, _CHUNK, _D), jnp.float32),
        pltpu.VMEM((_BPW,), jnp.int32),
        pltpu.SemaphoreType.DMA,
        pltpu.SemaphoreType.DMA,
        pltpu.SemaphoreType.DMA,
        pltpu.SemaphoreType.DMA,
    ],
)
def _sc_gather(table, idx, out_img, out_lab, idx_v, buf, lab_v,
               gsem0, gsem1, ssem0, ssem1):
    wid = lax.axis_index("s") * _NC + lax.axis_index("c")
    base = wid * _BPW
    gsems = (gsem0, gsem1)
    ssems = (ssem0, ssem1)

    pltpu.sync_copy(idx.at[pl.ds(base, _BPW)], idx_v)

    iota = lax.iota(jnp.int32, _L)
    for j in range(_BPW // _L):
        off = lax.broadcast_in_dim(base + j * _L, (_L,), ())
        lab_v[pl.ds(j * _L, _L)] = lax.shift_right_logical(
            iota + off, lax.broadcast_in_dim(jnp.int32(8), (_L,), ()))
    pltpu.sync_copy(lab_v, out_lab.at[pl.ds(base, _BPW)])

    def g_copy(j, b):
        return pltpu.make_async_copy(
            table.at[idx_v.at[pl.ds(j * _CHUNK, _CHUNK)]], buf.at[b], gsems[b])

    def s_copy(j, b):
        return pltpu.make_async_copy(
            buf.at[b], out_img.at[pl.ds(base + j * _CHUNK, _CHUNK)], ssems[b])

    g_copy(0, 0).start()
    for j in range(_NCHUNK):
        b = j % 2
        if j + 1 < _NCHUNK:
            if j >= 1:
                s_copy(j - 1, 1 - b).wait()
            g_copy(j + 1, 1 - b).start()
        g_copy(j, b).wait()
        s_copy(j, b).start()
    s_copy(_NCHUNK - 2, (_NCHUNK - 2) % 2).wait()
    s_copy(_NCHUNK - 1, (_NCHUNK - 1) % 2).wait()


def kernel(images_all, indices_class):
    table = images_all.reshape(_N_IMAGES, _D)
    idx = indices_class[:, :_BATCH].reshape(-1)
    imgs, labs = _sc_gather(table, idx)
    return imgs.reshape(_B, 3, 32, 32), labs
